# baseline (speedup 1.0000x reference)
import functools

import jax
import jax.numpy as jnp
import numpy as np
from jax.experimental import pallas as pl
from jax.experimental.pallas import tpu as pltpu

B, N, G, C = 8, 20000, 64, 80
FG_THRESH, BG_THRESH = 0.5, 0.4
ALPHA, GAMMA = 0.25, 2.0
EPS = 1e-7
DW_CLAMP = float(np.log(1000.0 / 16.0))

BLK = 20000
NB = N // BLK


def _atan_pos(z):
    t = jnp.minimum(z, 1.0 / z)
    t2 = t * t
    p = -0.0040540580
    p = p * t2 + 0.0218612288
    p = p * t2 - 0.0559098861
    p = p * t2 + 0.0964200441
    p = p * t2 - 0.1390853351
    p = p * t2 + 0.1994653599
    p = p * t2 - 0.3332985605
    p = p * t2 + 0.9999993329
    p = p * t
    return jnp.where(z <= 1.0, p, (np.pi / 2.0) - p)


def _sig_parts(x):
    t = jnp.exp(-jnp.abs(x))
    l = jnp.log(1.0 + t)
    r = 1.0 / (1.0 + t)
    p = jnp.where(x >= 0.0, r, t * r)
    ls_pos = jnp.minimum(x, 0.0) - l
    ls_neg = jnp.minimum(-x, 0.0) - l
    return p, ls_pos, ls_neg


def _iou_tile(ancT, tb):
    ax1, ay1, ax2, ay2 = ancT[0:1, :], ancT[1:2, :], ancT[2:3, :], ancT[3:4, :]
    tx1, ty1, tx2, ty2 = tb[:, 0:1], tb[:, 1:2], tb[:, 2:3], tb[:, 3:4]
    area_ae = (ax2 - ax1) * (ay2 - ay1) + EPS
    area_t = (tx2 - tx1) * (ty2 - ty1)
    iw = jnp.maximum(jnp.minimum(ax2, tx2) - jnp.maximum(ax1, tx1), 0.0)
    ih = jnp.maximum(jnp.minimum(ay2, ty2) - jnp.maximum(ay1, ty1), 0.0)
    inter = iw * ih
    return inter / ((area_t + area_ae) - inter)


def _dot00(a, b):
    return jax.lax.dot_general(a, b, (((0,), (0,)), ((), ())),
                               preferred_element_type=jnp.float32)


def _dot11(a, b):
    return jax.lax.dot_general(a, b, (((1,), (1,)), ((), ())),
                               preferred_element_type=jnp.float32)


def _fused_kernel(tb_ref, tlc_ref, ancT_ref, pbT_ref, plab_ref,
                  out_ref, hi_ref, acc_ref, iou_ref, mvam_ref):
    b = pl.program_id(0)
    p = pl.program_id(1)
    j = pl.program_id(2)
    tb = tb_ref[0]
    ancT = ancT_ref[0, 0]

    @pl.when(p == 0)
    def _():
        iou = _iou_tile(ancT, tb)
        iou_ref[pl.ds(j, 1)] = iou[None]
        part = jnp.max(iou, axis=1, keepdims=True)
        mv = jnp.max(iou, axis=0, keepdims=True)
        g_iota = jax.lax.broadcasted_iota(jnp.int32, (G, BLK), 0).astype(
            jnp.float32)
        am = jnp.min(jnp.where(iou == mv, g_iota, float(G)), axis=0,
                     keepdims=True)
        mvam_ref[0:1, :] = mv
        mvam_ref[1:2, :] = am

        @pl.when(j == 0)
        def _():
            hi_ref[...] = part

        @pl.when(j != 0)
        def _():
            hi_ref[...] = jnp.maximum(hi_ref[...], part)

    @pl.when(p == 1)
    def _():
        iou = iou_ref[pl.ds(j, 1)][0]
        mv = mvam_ref[0:1, :]
        am = mvam_ref[1:2, :]
        g_iota = jax.lax.broadcasted_iota(jnp.int32, (G, BLK), 0).astype(
            jnp.float32)

        m = jnp.where(mv < BG_THRESH, -1.0, am)
        m = jnp.where((mv >= BG_THRESH) & (mv < FG_THRESH), -2.0, m)
        force = jnp.max((iou == hi_ref[...]).astype(jnp.float32), axis=0,
                        keepdims=True) > 0.0
        m = jnp.where(force, am, m)

        fg = (m >= 0.0).astype(jnp.float32)
        valid = (m != -2.0).astype(jnp.float32)
        midx = jnp.maximum(m, 0.0)
        onehot = (g_iota == midx).astype(jnp.float32)

        gc = _dot00(tb, onehot)
        gx1, gy1, gx2, gy2 = gc[0:1, :], gc[1:2, :], gc[2:3, :], gc[3:4, :]

        tlc = tlc_ref[0]
        cg_iota = jax.lax.broadcasted_iota(jnp.int32, (G, C), 1).astype(
            jnp.float32)
        L = (tlc == cg_iota).astype(jnp.float32)
        sel = _dot00(onehot, L)

        x = plab_ref[0]
        pr, _, ls_neg = _sig_parts(x)
        bg = (-ls_neg) * pr * pr
        w_c = jnp.full((1, C), 1.0 - ALPHA, jnp.float32)
        bg_row = _dot11(w_c, bg)
        ones_c = jnp.ones((1, C), jnp.float32)
        xs = _dot11(ones_c, x * sel)

        ps, ls_pos_s, ls_neg_s = _sig_parts(xs)
        foc_fg = ALPHA * (-ls_pos_s) * (1.0 - ps) * (1.0 - ps)
        foc_bg = (1.0 - ALPHA) * (-ls_neg_s) * ps * ps
        cls_part = jnp.sum(valid * bg_row + fg * (foc_fg - foc_bg),
                           axis=(0, 1), keepdims=True)

        ax1, ay1, ax2, ay2 = (ancT[0:1, :], ancT[1:2, :], ancT[2:3, :],
                              ancT[3:4, :])
        aw = ax2 - ax1
        ah = ay2 - ay1
        acx = ax1 + 0.5 * aw
        acy = ay1 + 0.5 * ah
        pbT = pbT_ref[0, 0]
        dx, dy = pbT[0:1, :], pbT[1:2, :]
        dw = jnp.minimum(pbT[2:3, :], DW_CLAMP)
        dh = jnp.minimum(pbT[3:4, :], DW_CLAMP)
        pcx = dx * aw + acx
        pcy = dy * ah + acy
        pw2 = 0.5 * jnp.exp(dw) * aw
        ph2 = 0.5 * jnp.exp(dh) * ah
        px1, px2 = pcx - pw2, pcx + pw2
        py1, py2 = pcy - ph2, pcy + ph2

        iw = jnp.maximum(jnp.minimum(px2, gx2) - jnp.maximum(px1, gx1), 0.0)
        ih = jnp.maximum(jnp.minimum(py2, gy2) - jnp.maximum(py1, gy1), 0.0)
        inter = iw * ih
        pw = px2 - px1
        ph = py2 - py1
        gw = gx2 - gx1
        gh = gy2 - gy1
        union = pw * ph + gw * gh - inter
        biou = inter / (union + EPS)
        cw = jnp.maximum(px2, gx2) - jnp.minimum(px1, gx1)
        ch = jnp.maximum(py2, gy2) - jnp.minimum(py1, gy1)
        c2 = cw * cw + ch * ch + EPS
        rho2 = ((pcx - (gx1 + gx2) * 0.5) ** 2
                + (pcy - (gy1 + gy2) * 0.5) ** 2)
        ge = gh + EPS
        pe = ph + EPS
        au = jnp.abs((gw * pe - pw * ge) / (ge * pe + gw * pw))
        v = (4.0 / (np.pi ** 2)) * _atan_pos(au) ** 2
        alpha = v / (1.0 - biou + v + EPS)
        bl = 1.0 - (biou - rho2 / c2 - alpha * v)
        box_part = jnp.sum(fg * bl, axis=(0, 1), keepdims=True)
        nfg_part = jnp.sum(fg, axis=(0, 1), keepdims=True)

        row = jnp.concatenate([cls_part, box_part, nfg_part], axis=1)

        @pl.when(j == 0)
        def _():
            acc_ref[pl.ds(b, 1), 0:3] = row

        @pl.when(j != 0)
        def _():
            acc_ref[pl.ds(b, 1), 0:3] += row

        @pl.when((b == B - 1) & (j == NB - 1))
        def _():
            cls_c = acc_ref[:, 0:1]
            box_c = acc_ref[:, 1:2]
            nfg_c = acc_ref[:, 2:3]
            denom = jnp.maximum(1.0, nfg_c)
            clsm = jnp.sum(cls_c / denom, axis=(0, 1), keepdims=True) / B
            boxm = jnp.sum(box_c / denom, axis=(0, 1), keepdims=True) / B
            out_ref[0:1, 0:1] = clsm
            out_ref[0:1, 1:2] = boxm


@functools.partial(jax.jit, static_argnames=("interpret",))
def kernel(pred_boxes, pred_labels, target_boxes, target_labels, anchors,
           interpret=False):
    def _coord_planes(a):
        return jnp.transpose(
            jnp.transpose(a, (0, 2, 1)).reshape(B, 4, NB, BLK), (0, 2, 1, 3))

    ancT = _coord_planes(anchors)
    pbT = _coord_planes(pred_boxes)
    tl_col = target_labels.astype(jnp.float32)[:, :, None]

    out = pl.pallas_call(
        _fused_kernel,
        grid=(B, 2, NB),
        in_specs=[
            pl.BlockSpec((1, G, 4), lambda b, p, j: (b, 0, 0)),
            pl.BlockSpec((1, G, 1), lambda b, p, j: (b, 0, 0)),
            pl.BlockSpec((1, 1, 4, BLK), lambda b, p, j: (b, j, 0, 0)),
            pl.BlockSpec((1, 1, 4, BLK), lambda b, p, j: (b, j * p, 0, 0)),
            pl.BlockSpec((1, BLK, C), lambda b, p, j: (b, j * p, 0)),
        ],
        out_specs=pl.BlockSpec((1, 2), lambda b, p, j: (0, 0)),
        out_shape=jax.ShapeDtypeStruct((1, 2), jnp.float32),
        scratch_shapes=[
            pltpu.VMEM((G, 1), jnp.float32),
            pltpu.VMEM((B, 128), jnp.float32),
            pltpu.VMEM((NB, G, BLK), jnp.float32),
            pltpu.VMEM((2, BLK), jnp.float32),
        ],
        interpret=interpret,
    )(target_boxes, tl_col, ancT, pbT, pred_labels)

    return out[0]


# speedup vs baseline: 1.0120x; 1.0120x over previous
import functools

import jax
import jax.numpy as jnp
import numpy as np
from jax.experimental import pallas as pl
from jax.experimental.pallas import tpu as pltpu

B, N, G, C = 8, 20000, 64, 80
FG_THRESH, BG_THRESH = 0.5, 0.4
ALPHA, GAMMA = 0.25, 2.0
EPS = 1e-7
DW_CLAMP = float(np.log(1000.0 / 16.0))

BLK = 20000
NB = N // BLK


def _atan_pos(z):
    t = jnp.minimum(z, 1.0 / z)
    t2 = t * t
    p = -0.0040540580
    p = p * t2 + 0.0218612288
    p = p * t2 - 0.0559098861
    p = p * t2 + 0.0964200441
    p = p * t2 - 0.1390853351
    p = p * t2 + 0.1994653599
    p = p * t2 - 0.3332985605
    p = p * t2 + 0.9999993329
    p = p * t
    return jnp.where(z <= 1.0, p, (np.pi / 2.0) - p)


def _sig_parts(x):
    t = jnp.exp(-jnp.abs(x))
    l = jnp.log(1.0 + t)
    r = 1.0 / (1.0 + t)
    p = jnp.where(x >= 0.0, r, t * r)
    ls_pos = jnp.minimum(x, 0.0) - l
    ls_neg = jnp.minimum(-x, 0.0) - l
    return p, ls_pos, ls_neg


def _iou_tile(ancT, tb):
    ax1, ay1, ax2, ay2 = ancT[0:1, :], ancT[1:2, :], ancT[2:3, :], ancT[3:4, :]
    tx1, ty1, tx2, ty2 = tb[:, 0:1], tb[:, 1:2], tb[:, 2:3], tb[:, 3:4]
    area_ae = (ax2 - ax1) * (ay2 - ay1) + EPS
    area_t = (tx2 - tx1) * (ty2 - ty1)
    iw = jnp.maximum(jnp.minimum(ax2, tx2) - jnp.maximum(ax1, tx1), 0.0)
    ih = jnp.maximum(jnp.minimum(ay2, ty2) - jnp.maximum(ay1, ty1), 0.0)
    inter = iw * ih
    return inter / ((area_t + area_ae) - inter)


def _dot00(a, b):
    return jax.lax.dot_general(a, b, (((0,), (0,)), ((), ())),
                               preferred_element_type=jnp.float32)


def _dot11(a, b):
    return jax.lax.dot_general(a, b, (((1,), (1,)), ((), ())),
                               preferred_element_type=jnp.float32)


def _fused_kernel(tb_ref, tlc_ref, ancT_ref, pbT_ref, plab_ref,
                  out_ref, hi_ref, acc_ref, iou_ref):
    b = pl.program_id(0)
    p = pl.program_id(1)
    j = pl.program_id(2)
    tb = tb_ref[0]
    ancT = ancT_ref[0, 0]

    @pl.when(p == 0)
    def _():
        iou = _iou_tile(ancT, tb)
        iou_ref[pl.ds(j, 1)] = iou[None]
        part = jnp.max(iou, axis=1, keepdims=True)

        @pl.when(j == 0)
        def _():
            hi_ref[...] = part

        @pl.when(j != 0)
        def _():
            hi_ref[...] = jnp.maximum(hi_ref[...], part)

    @pl.when(p == 1)
    def _():
        iou = iou_ref[pl.ds(j, 1)][0]
        mv = jnp.max(iou, axis=0, keepdims=True)
        g_iota = jax.lax.broadcasted_iota(jnp.int32, (G, BLK), 0).astype(
            jnp.float32)
        am = jnp.min(jnp.where(iou == mv, g_iota, float(G)), axis=0,
                     keepdims=True)

        m = jnp.where(mv < BG_THRESH, -1.0, am)
        m = jnp.where((mv >= BG_THRESH) & (mv < FG_THRESH), -2.0, m)
        force = jnp.max((iou == hi_ref[...]).astype(jnp.float32), axis=0,
                        keepdims=True) > 0.0
        m = jnp.where(force, am, m)

        fg = (m >= 0.0).astype(jnp.float32)
        valid = (m != -2.0).astype(jnp.float32)
        midx = jnp.maximum(m, 0.0)
        onehot = (g_iota == midx).astype(jnp.float32)

        gc = _dot00(tb, onehot)
        gx1, gy1, gx2, gy2 = gc[0:1, :], gc[1:2, :], gc[2:3, :], gc[3:4, :]

        tlc = tlc_ref[0]
        cg_iota = jax.lax.broadcasted_iota(jnp.int32, (G, C), 1).astype(
            jnp.float32)
        L = (tlc == cg_iota).astype(jnp.float32)
        sel = _dot00(onehot, L)

        x = plab_ref[0]
        pr, _, ls_neg = _sig_parts(x)
        bg = (-ls_neg) * pr * pr
        w_c = jnp.full((1, C), 1.0 - ALPHA, jnp.float32)
        bg_row = _dot11(w_c, bg)
        ones_c = jnp.ones((1, C), jnp.float32)
        xs = _dot11(ones_c, x * sel)

        ps, ls_pos_s, ls_neg_s = _sig_parts(xs)
        foc_fg = ALPHA * (-ls_pos_s) * (1.0 - ps) * (1.0 - ps)
        foc_bg = (1.0 - ALPHA) * (-ls_neg_s) * ps * ps
        cls_part = jnp.sum(valid * bg_row + fg * (foc_fg - foc_bg),
                           axis=(0, 1), keepdims=True)

        ax1, ay1, ax2, ay2 = (ancT[0:1, :], ancT[1:2, :], ancT[2:3, :],
                              ancT[3:4, :])
        aw = ax2 - ax1
        ah = ay2 - ay1
        acx = ax1 + 0.5 * aw
        acy = ay1 + 0.5 * ah
        pbT = pbT_ref[0, 0]
        dx, dy = pbT[0:1, :], pbT[1:2, :]
        dw = jnp.minimum(pbT[2:3, :], DW_CLAMP)
        dh = jnp.minimum(pbT[3:4, :], DW_CLAMP)
        pcx = dx * aw + acx
        pcy = dy * ah + acy
        pw2 = 0.5 * jnp.exp(dw) * aw
        ph2 = 0.5 * jnp.exp(dh) * ah
        px1, px2 = pcx - pw2, pcx + pw2
        py1, py2 = pcy - ph2, pcy + ph2

        iw = jnp.maximum(jnp.minimum(px2, gx2) - jnp.maximum(px1, gx1), 0.0)
        ih = jnp.maximum(jnp.minimum(py2, gy2) - jnp.maximum(py1, gy1), 0.0)
        inter = iw * ih
        pw = px2 - px1
        ph = py2 - py1
        gw = gx2 - gx1
        gh = gy2 - gy1
        union = pw * ph + gw * gh - inter
        biou = inter / (union + EPS)
        cw = jnp.maximum(px2, gx2) - jnp.minimum(px1, gx1)
        ch = jnp.maximum(py2, gy2) - jnp.minimum(py1, gy1)
        c2 = cw * cw + ch * ch + EPS
        rho2 = ((pcx - (gx1 + gx2) * 0.5) ** 2
                + (pcy - (gy1 + gy2) * 0.5) ** 2)
        ge = gh + EPS
        pe = ph + EPS
        au = jnp.abs((gw * pe - pw * ge) / (ge * pe + gw * pw))
        v = (4.0 / (np.pi ** 2)) * _atan_pos(au) ** 2
        alpha = v / (1.0 - biou + v + EPS)
        bl = 1.0 - (biou - rho2 / c2 - alpha * v)
        box_part = jnp.sum(fg * bl, axis=(0, 1), keepdims=True)
        nfg_part = jnp.sum(fg, axis=(0, 1), keepdims=True)

        row = jnp.concatenate([cls_part, box_part, nfg_part], axis=1)

        @pl.when(j == 0)
        def _():
            acc_ref[pl.ds(b, 1), 0:3] = row

        @pl.when(j != 0)
        def _():
            acc_ref[pl.ds(b, 1), 0:3] += row

        @pl.when((b == B - 1) & (j == NB - 1))
        def _():
            cls_c = acc_ref[:, 0:1]
            box_c = acc_ref[:, 1:2]
            nfg_c = acc_ref[:, 2:3]
            denom = jnp.maximum(1.0, nfg_c)
            clsm = jnp.sum(cls_c / denom, axis=(0, 1), keepdims=True) / B
            boxm = jnp.sum(box_c / denom, axis=(0, 1), keepdims=True) / B
            out_ref[0:1, 0:1] = clsm
            out_ref[0:1, 1:2] = boxm


@functools.partial(jax.jit, static_argnames=("interpret",))
def kernel(pred_boxes, pred_labels, target_boxes, target_labels, anchors,
           interpret=False):
    def _coord_planes(a):
        return jnp.transpose(
            jnp.transpose(a, (0, 2, 1)).reshape(B, 4, NB, BLK), (0, 2, 1, 3))

    ancT = _coord_planes(anchors)
    pbT = _coord_planes(pred_boxes)
    tl_col = target_labels.astype(jnp.float32)[:, :, None]

    out = pl.pallas_call(
        _fused_kernel,
        grid=(B, 2, NB),
        in_specs=[
            pl.BlockSpec((1, G, 4), lambda b, p, j: (b, 0, 0)),
            pl.BlockSpec((1, G, 1), lambda b, p, j: (b, 0, 0)),
            pl.BlockSpec((1, 1, 4, BLK), lambda b, p, j: (b, j, 0, 0)),
            pl.BlockSpec((1, 1, 4, BLK), lambda b, p, j: (b, j * p, 0, 0)),
            pl.BlockSpec((1, BLK, C), lambda b, p, j: (b, j * p, 0)),
        ],
        out_specs=pl.BlockSpec((1, 2), lambda b, p, j: (0, 0)),
        out_shape=jax.ShapeDtypeStruct((1, 2), jnp.float32),
        scratch_shapes=[
            pltpu.VMEM((G, 1), jnp.float32),
            pltpu.VMEM((B, 128), jnp.float32),
            pltpu.VMEM((NB, G, BLK), jnp.float32),
        ],
        interpret=interpret,
    )(target_boxes, tl_col, ancT, pbT, pred_labels)

    return out[0]
